# baseline (speedup 1.0000x reference)
import jax
import jax.numpy as jnp
from jax.experimental import pallas as pl


def _ln(x, g, b):
    m = x.mean(-1, keepdims=True)
    v = x.var(-1, keepdims=True)
    return (x - m) / jnp.sqrt(v + 1e-5) * g + b


def _add_kernel(a_ref, b_ref, o_ref):
    o_ref[...] = a_ref[...] + b_ref[...]


def _padd(a, b):
    return pl.pallas_call(
        _add_kernel,
        out_shape=jax.ShapeDtypeStruct(a.shape, a.dtype),
    )(a, b)


def kernel(x, edge_index, edge_attr, ln1_g, ln1_b, Wq, bq, Wk, bk, Wv, bv,
           We, W_skip, b_skip, w_beta, ln2_g, ln2_b, W1, b1, W2, b2):
    n = x.shape[0]
    H, C = 8, 16
    src = edge_index[0]
    dst = edge_index[1]
    h = _ln(x, ln1_g, ln1_b)
    q = (h @ Wq + bq).reshape(n, H, C)
    k = (h @ Wk + bk).reshape(n, H, C)
    v = (h @ Wv + bv).reshape(n, H, C)
    e = (edge_attr @ We).reshape(-1, H, C)
    kj = k[src] + e
    alpha = (q[dst] * kj).sum(-1) / jnp.sqrt(float(C))
    amax = jax.ops.segment_max(alpha, dst, num_segments=n)
    amax = jnp.where(jnp.isfinite(amax), amax, 0.0)
    ex = jnp.exp(alpha - amax[dst])
    den = jax.ops.segment_sum(ex, dst, num_segments=n)
    w = ex / (den[dst] + 1e-16)
    msg = (v[src] + e) * w[:, :, None]
    out = jax.ops.segment_sum(msg, dst, num_segments=n).reshape(n, -1)
    x_skip = h @ W_skip + b_skip
    g = jax.nn.sigmoid(jnp.concatenate([out, x_skip, out - x_skip], axis=-1) @ w_beta)
    out = g * x_skip + (1.0 - g) * out
    x1 = _padd(x, out)
    h2 = _ln(x1, ln2_g, ln2_b)
    f = jax.nn.gelu(h2 @ W1 + b1, approximate=False) @ W2 + b2
    return _padd(x1, f)


# speedup vs baseline: 17.5445x; 17.5445x over previous
import functools

import jax
import jax.numpy as jnp
from jax import lax
from jax.experimental import pallas as pl
from jax.experimental.pallas import tpu as pltpu
from jax.experimental.pallas import tpu_sc as plsc

N = 10000
E = 320000
D = 128
H = 8
C = 16
ED = 16
FF = 2 * D

NC = 2
NS = 16
NW = NC * NS
EPW = E // NW
BE = 40
NBLK = EPW // BE
RPT = 624

RB = 1000
GRID = N // RB


def _ln_rows(x, g, b):
    m = x.mean(-1, keepdims=True)
    v = ((x - m) ** 2).mean(-1, keepdims=True)
    return (x - m) / jnp.sqrt(v + 1e-5) * g + b


def _stage_a_body(x_ref, g_ref, b_ref, wq_ref, bq_ref, wk_ref, bk_ref,
                  wv_ref, bv_ref, bqd_ref, ws_ref, bs_ref,
                  qt_ref, kv_ref, xs_ref):
    h = _ln_rows(x_ref[...], g_ref[...], b_ref[...])
    q = jnp.dot(h, wq_ref[...], preferred_element_type=jnp.float32) + bq_ref[...]
    qw = jnp.dot(q, bqd_ref[...], preferred_element_type=jnp.float32)
    qt_ref[:, :D] = q
    qt_ref[:, D:] = qw
    kv_ref[:, :D] = jnp.dot(h, wk_ref[...], preferred_element_type=jnp.float32) + bk_ref[...]
    kv_ref[:, D:] = jnp.dot(h, wv_ref[...], preferred_element_type=jnp.float32) + bv_ref[...]
    xs_ref[...] = jnp.dot(h, ws_ref[...], preferred_element_type=jnp.float32) + bs_ref[...]


def _stage_a(x, ln1_g, ln1_b, Wq, bq, Wk, bk, Wv, bv, Bq, W_skip, b_skip):
    row = lambda i: (i, 0)
    full = lambda i: (0, 0)
    wspec = pl.BlockSpec((D, D), full)
    vspec = pl.BlockSpec((1, D), full)
    return pl.pallas_call(
        _stage_a_body,
        grid=(GRID,),
        in_specs=[pl.BlockSpec((RB, D), row), vspec, vspec,
                  wspec, vspec, wspec, vspec, wspec, vspec,
                  wspec, wspec, vspec],
        out_specs=[pl.BlockSpec((RB, 2 * D), row),
                   pl.BlockSpec((RB, 2 * D), row),
                   pl.BlockSpec((RB, D), row)],
        out_shape=[jax.ShapeDtypeStruct((N, 2 * D), jnp.float32),
                   jax.ShapeDtypeStruct((N, 2 * D), jnp.float32),
                   jax.ShapeDtypeStruct((N, D), jnp.float32)],
    )(x, ln1_g.reshape(1, D), ln1_b.reshape(1, D), Wq, bq.reshape(1, D),
      Wk, bk.reshape(1, D), Wv, bv.reshape(1, D), Bq, W_skip,
      b_skip.reshape(1, D))


def _edge1_body(qt_hbm, kv_hbm, src_hbm, dst_hbm, ea_hbm, z128_hbm, z16_hbm,
                out1_hbm, den_hbm, p_hbm,
                out_sh, den_sh, src_v, dst_v, qt_v, kv_v, ea_v, msg_v, p_v):
    cid = lax.axis_index("c")
    sid = lax.axis_index("s")
    wid = cid * NS + sid
    lanes = lax.broadcasted_iota(jnp.int32, (16,), 0)

    r0z = sid * RPT
    pltpu.sync_copy(z128_hbm.at[pl.ds(r0z, RPT)], out_sh.at[pl.ds(r0z, RPT)])
    pltpu.sync_copy(z16_hbm.at[pl.ds(r0z, RPT)], den_sh.at[pl.ds(r0z, RPT)])

    @pl.when(sid == 0)
    def _zero_tail():
        t0 = NS * RPT
        pltpu.sync_copy(z128_hbm.at[pl.ds(t0, N - t0)],
                        out_sh.at[pl.ds(t0, N - t0)])
        pltpu.sync_copy(z16_hbm.at[pl.ds(t0, N - t0)],
                        den_sh.at[pl.ds(t0, N - t0)])

    plsc.subcore_barrier()

    @pl.loop(0, NBLK)
    def _block(blk):
        eb = wid * EPW + blk * BE
        pltpu.sync_copy(src_hbm.at[pl.ds(eb, BE)], src_v)
        pltpu.sync_copy(dst_hbm.at[pl.ds(eb, BE)], dst_v)
        pltpu.sync_copy(ea_hbm.at[pl.ds(eb, BE)], ea_v)
        pltpu.sync_copy(qt_hbm.at[dst_v], qt_v)
        pltpu.sync_copy(kv_hbm.at[src_v], kv_v)

        @pl.loop(0, BE)
        def _edge(r):
            ea_row = ea_v[r, pl.ds(0, ED)]
            alphas = jnp.zeros((16,), jnp.float32)
            for h in range(H):
                qh = qt_v[r, pl.ds(h * C, C)]
                kh = kv_v[r, pl.ds(h * C, C)]
                qwh = qt_v[r, pl.ds(D + h * C, C)]
                a = jnp.sum(qh * kh + qwh * ea_row) * 0.25
                alphas = jnp.where(lanes == h, jnp.full((16,), a, jnp.float32),
                                   alphas)
            p_row = jnp.exp(jnp.clip(alphas, -60.0, 60.0))
            for h in range(H):
                ph = jnp.full((16,), p_row[h], jnp.float32)
                vh = kv_v[r, pl.ds(D + h * C, C)]
                msg_v[r, pl.ds(h * C, C)] = vh * ph
            p_v[r, pl.ds(0, 16)] = p_row

        pltpu.sync_copy(msg_v, out_sh.at[dst_v], add=True)
        pltpu.sync_copy(p_v, den_sh.at[dst_v], add=True)
        pltpu.sync_copy(p_v, p_hbm.at[pl.ds(eb, BE)])

    plsc.subcore_barrier()
    r0 = sid * RPT
    pltpu.sync_copy(out_sh.at[pl.ds(r0, RPT)], out1_hbm.at[cid, pl.ds(r0, RPT)])
    pltpu.sync_copy(den_sh.at[pl.ds(r0, RPT)], den_hbm.at[cid, pl.ds(r0, RPT)])

    @pl.when(sid == 0)
    def _copy_tail():
        t0 = NS * RPT
        pltpu.sync_copy(out_sh.at[pl.ds(t0, N - t0)],
                        out1_hbm.at[cid, pl.ds(t0, N - t0)])
        pltpu.sync_copy(den_sh.at[pl.ds(t0, N - t0)],
                        den_hbm.at[cid, pl.ds(t0, N - t0)])


def _edge_pass1(qt, kv, src, dst, edge_attr):
    mesh = plsc.VectorSubcoreMesh(core_axis_name="c", subcore_axis_name="s",
                                  num_cores=NC, num_subcores=NS)
    params = pltpu.CompilerParams(needs_layout_passes=False,
                                  use_tc_tiling_on_sc=False)
    f = pl.kernel(
        _edge1_body,
        out_type=[jax.ShapeDtypeStruct((NC, N, D), jnp.float32),
                  jax.ShapeDtypeStruct((NC, N, 16), jnp.float32),
                  jax.ShapeDtypeStruct((E, 16), jnp.float32)],
        mesh=mesh,
        compiler_params=params,
        scratch_types=[
            pltpu.VMEM_SHARED((N, D), jnp.float32),
            pltpu.VMEM_SHARED((N, 16), jnp.float32),
            pltpu.VMEM((BE,), jnp.int32),
            pltpu.VMEM((BE,), jnp.int32),
            pltpu.VMEM((BE, 2 * D), jnp.float32),
            pltpu.VMEM((BE, 2 * D), jnp.float32),
            pltpu.VMEM((BE, ED), jnp.float32),
            pltpu.VMEM((BE, D), jnp.float32),
            pltpu.VMEM((BE, 16), jnp.float32),
        ],
    )
    z128 = jnp.zeros((N, D), jnp.float32)
    z16 = jnp.zeros((N, 16), jnp.float32)
    return f(qt, kv, src, dst, edge_attr, z128, z16)


def _edge2_body(dst_hbm, ea_hbm, p_hbm, t_hbm,
                t_sh, dst_v, ea_v, p_v, t_v):
    cid = lax.axis_index("c")
    sid = lax.axis_index("s")
    wid = cid * NS + sid
    zvec = jnp.zeros((16,), jnp.float32)
    lanes = lax.broadcasted_iota(jnp.int32, (16,), 0)

    for r in range(BE):
        for j in range(D // 16):
            t_v[r, pl.ds(j * 16, 16)] = zvec
    for kk in range((RPT + BE - 1) // BE):
        start = sid * RPT + min(kk * BE, RPT - BE)
        pltpu.sync_copy(t_v, t_sh.at[pl.ds(start, BE)])

    @pl.when(sid == 0)
    def _zero_tail():
        pltpu.sync_copy(t_v, t_sh.at[pl.ds(N - BE, BE)])

    plsc.subcore_barrier()

    @pl.loop(0, NBLK)
    def _block(blk):
        eb = wid * EPW + blk * BE
        pltpu.sync_copy(dst_hbm.at[pl.ds(eb, BE)], dst_v)
        pltpu.sync_copy(ea_hbm.at[pl.ds(eb, BE)], ea_v)
        pltpu.sync_copy(p_hbm.at[pl.ds(eb, BE)], p_v)

        @pl.loop(0, BE)
        def _edge(r):
            ea_row = ea_v[r, pl.ds(0, ED)]
            p_row = p_v[r, pl.ds(0, 16)]
            for h in range(H):
                ph = jnp.full((16,), p_row[h], jnp.float32)
                t_v[r, pl.ds(h * C, C)] = ph * ea_row

        pltpu.sync_copy(t_v, t_sh.at[dst_v], add=True)

    plsc.subcore_barrier()
    r0 = sid * RPT
    pltpu.sync_copy(t_sh.at[pl.ds(r0, RPT)], t_hbm.at[cid, pl.ds(r0, RPT)])

    @pl.when(sid == 0)
    def _copy_tail():
        t0 = NS * RPT
        pltpu.sync_copy(t_sh.at[pl.ds(t0, N - t0)],
                        t_hbm.at[cid, pl.ds(t0, N - t0)])


def _edge_pass2(dst, edge_attr, p):
    mesh = plsc.VectorSubcoreMesh(core_axis_name="c", subcore_axis_name="s",
                                  num_cores=NC, num_subcores=NS)
    params = pltpu.CompilerParams(needs_layout_passes=False,
                                  use_tc_tiling_on_sc=False)
    f = pl.kernel(
        _edge2_body,
        out_type=jax.ShapeDtypeStruct((NC, N, D), jnp.float32),
        mesh=mesh,
        compiler_params=params,
        scratch_types=[
            pltpu.VMEM_SHARED((N, D), jnp.float32),
            pltpu.VMEM((BE,), jnp.int32),
            pltpu.VMEM((BE, ED), jnp.float32),
            pltpu.VMEM((BE, 16), jnp.float32),
            pltpu.VMEM((BE, D), jnp.float32),
        ],
    )
    return f(dst, edge_attr, p)


def _stage_c_body(x_ref, out1_ref, den_ref, t_ref, xs_ref, b2_ref, erep_ref,
                  wba_ref, wbb_ref, g2_ref, bb2_ref, w1_ref, b1_ref,
                  w2_ref, bf2_ref, y_ref):
    out1 = out1_ref[0] + out1_ref[1]
    den = den_ref[0, :, :H] + den_ref[1, :, :H]
    t = t_ref[0] + t_ref[1]
    out2 = jnp.dot(t, b2_ref[...], preferred_element_type=jnp.float32)
    deninv = 1.0 / (den + 1e-16)
    scale = jnp.dot(deninv, erep_ref[...], preferred_element_type=jnp.float32)
    att = (out1 + out2) * scale
    xs = xs_ref[...]
    gdot = jnp.sum(att * wba_ref[...] + xs * wbb_ref[...], axis=-1, keepdims=True)
    g = jax.nn.sigmoid(gdot)
    outg = g * xs + (1.0 - g) * att
    x1 = x_ref[...] + outg
    h2 = _ln_rows(x1, g2_ref[...], bb2_ref[...])
    u = jnp.dot(h2, w1_ref[...], preferred_element_type=jnp.float32) + b1_ref[...]
    f = u * 0.5 * (1.0 + lax.erf(u * 0.7071067811865476))
    f = jnp.dot(f, w2_ref[...], preferred_element_type=jnp.float32) + bf2_ref[...]
    y_ref[...] = x1 + f


def _stage_c(x, out1p, denp, tp, xs, B2, Erep, wbA, wbB, ln2_g, ln2_b,
             W1, b1, W2, b2):
    row = lambda i: (i, 0)
    row3 = lambda i: (0, i, 0)
    full = lambda i: (0, 0)
    return pl.pallas_call(
        _stage_c_body,
        grid=(GRID,),
        in_specs=[pl.BlockSpec((RB, D), row),
                  pl.BlockSpec((NC, RB, D), row3),
                  pl.BlockSpec((NC, RB, 16), row3),
                  pl.BlockSpec((NC, RB, D), row3),
                  pl.BlockSpec((RB, D), row),
                  pl.BlockSpec((D, D), full),
                  pl.BlockSpec((H, D), full),
                  pl.BlockSpec((1, D), full),
                  pl.BlockSpec((1, D), full),
                  pl.BlockSpec((1, D), full),
                  pl.BlockSpec((1, D), full),
                  pl.BlockSpec((D, FF), full),
                  pl.BlockSpec((1, FF), full),
                  pl.BlockSpec((FF, D), full),
                  pl.BlockSpec((1, D), full)],
        out_specs=pl.BlockSpec((RB, D), row),
        out_shape=jax.ShapeDtypeStruct((N, D), jnp.float32),
    )(x, out1p, denp, tp, xs, B2, Erep, wbA, wbB,
      ln2_g.reshape(1, D), ln2_b.reshape(1, D), W1, b1.reshape(1, FF),
      W2, b2.reshape(1, D))


def kernel(x, edge_index, edge_attr, ln1_g, ln1_b, Wq, bq, Wk, bk, Wv, bv,
           We, W_skip, b_skip, w_beta, ln2_g, ln2_b, W1, b1, W2, b2):
    WeR = We.reshape(ED, H, C)
    B2 = jax.scipy.linalg.block_diag(*[WeR[:, h, :] for h in range(H)])
    Bq = B2.T
    Erep = jnp.repeat(jnp.eye(H, dtype=jnp.float32), C, axis=1)
    wbA = (w_beta[:D, 0] + w_beta[2 * D:, 0]).reshape(1, D)
    wbB = (w_beta[D:2 * D, 0] - w_beta[2 * D:, 0]).reshape(1, D)

    qt, kv, xs = _stage_a(x, ln1_g, ln1_b, Wq, bq, Wk, bk, Wv, bv, Bq,
                          W_skip, b_skip)
    src = edge_index[0]
    dst = edge_index[1]
    out1p, denp, p = _edge_pass1(qt, kv, src, dst, edge_attr)
    tp = _edge_pass2(dst, edge_attr, p)
    return _stage_c(x, out1p, denp, tp, xs, B2, Erep, wbA, wbB,
                    ln2_g, ln2_b, W1, b1, W2, b2)
